# baseline (speedup 1.0000x reference)
import functools

import jax
import jax.numpy as jnp
from jax import lax
from jax.experimental import pallas as pl
from jax.experimental.pallas import tpu as pltpu
from jax.experimental.pallas import tpu_sc as plsc

_NUM_BUCKETS = 1000000
_CHUNK = 128


@jax.jit
def _lookup(x, table):
    (batch,) = x.shape
    _, dim = table.shape
    info = plsc.get_sparse_core_info()
    num_cores, num_subcores, lanes = (
        info.num_cores, info.num_subcores, info.num_lanes)
    num_workers = num_cores * num_subcores
    b_per_w = batch // num_workers
    n_chunks = b_per_w // _CHUNK
    mesh = plsc.VectorSubcoreMesh(core_axis_name="c", subcore_axis_name="s")

    @functools.partial(
        pl.kernel,
        mesh=mesh,
        compiler_params=pltpu.CompilerParams(use_tc_tiling_on_sc=False),
        out_type=jax.ShapeDtypeStruct((batch, dim), jnp.float32),
        scratch_types=[
            pltpu.VMEM((b_per_w,), jnp.int32),
            pltpu.VMEM((b_per_w, dim), jnp.float32),
            pltpu.SemaphoreType.DMA,
        ],
    )
    def body(x_hbm, table_hbm, out_hbm, idx_v, rows_v, sem):
        wid = lax.axis_index("s") * num_cores + lax.axis_index("c")
        base = wid * b_per_w
        pltpu.sync_copy(x_hbm.at[pl.ds(base, b_per_w)], idx_v)
        for i in range(b_per_w // lanes):
            sl = pl.ds(i * lanes, lanes)
            idx_v[sl] = lax.rem(idx_v[sl], _NUM_BUCKETS)
        copies = [
            pltpu.async_copy(
                table_hbm.at[idx_v.at[pl.ds(c * _CHUNK, _CHUNK)]],
                rows_v.at[pl.ds(c * _CHUNK, _CHUNK)],
                sem,
            )
            for c in range(n_chunks)
        ]
        for cp in copies:
            cp.wait()
        pltpu.sync_copy(rows_v, out_hbm.at[pl.ds(base, b_per_w)])

    return body(x, table)


def kernel(x, table):
    return _lookup(x.astype(jnp.int32), table)


# speedup vs baseline: 1.5811x; 1.5811x over previous
import functools

import jax
import jax.numpy as jnp
from jax import lax
from jax.experimental import pallas as pl
from jax.experimental.pallas import tpu as pltpu
from jax.experimental.pallas import tpu_sc as plsc

_NUM_BUCKETS = 1000000


@jax.jit
def _lookup(x, table):
    (batch,) = x.shape
    nrows, dim = table.shape
    info = plsc.get_sparse_core_info()
    num_cores, num_subcores, lanes = (
        info.num_cores, info.num_subcores, info.num_lanes)
    num_workers = num_cores * num_subcores
    b_per_w = batch // num_workers
    grp = 32
    n_grp = b_per_w // grp
    mesh = plsc.VectorSubcoreMesh(core_axis_name="c", subcore_axis_name="s")

    @functools.partial(
        pl.kernel,
        mesh=mesh,
        compiler_params=pltpu.CompilerParams(needs_layout_passes=False),
        out_type=jax.ShapeDtypeStruct((dim, batch), jnp.float32),
        scratch_types=[
            pltpu.VMEM((b_per_w,), jnp.int32),
            pltpu.VMEM((2, grp, 8, dim), jnp.float32),
            pltpu.VMEM((dim, b_per_w), jnp.float32),
            pltpu.SemaphoreType.DMA,
        ],
    )
    def body(x_hbm, table_hbm, outT_hbm, hv, win, stgT, sem):
        wid = lax.axis_index("s") * num_cores + lax.axis_index("c")
        base = wid * b_per_w
        pltpu.sync_copy(x_hbm.at[pl.ds(base, b_per_w)], hv)
        for i in range(b_per_w // lanes):
            sl = pl.ds(i * lanes, lanes)
            hv[sl] = lax.rem(hv[sl], _NUM_BUCKETS)

        lane_iota = lax.iota(jnp.int32, lanes)

        def hval(k):
            dup = plsc.load_gather(hv, [jnp.full((lanes,), k, jnp.int32)])
            return lax.reduce_max(dup, axes=(0,))

        def fire(g, buf):
            def dma_body(r, _):
                h = hval(g * grp + r)
                h0 = (h // 8) * 8
                pltpu.async_copy(
                    table_hbm.at[pl.ds(h0, 8)], win.at[buf, r], sem)
                return 0
            lax.fori_loop(0, grp, dma_body, 0)

        def drain(buf):
            def wait_body(r, _):
                pltpu.make_async_copy(
                    table_hbm.at[pl.ds(0, 8)], win.at[buf, 0], sem).wait()
                return 0
            lax.fori_loop(0, grp, wait_body, 0)

        def extract(g, buf):
            def ex_body(r, _):
                dup = plsc.load_gather(
                    hv, [jnp.full((lanes,), g * grp + r, jnp.int32)])
                c = lax.rem(dup, 8)
                k = jnp.full((lanes,), g * grp + r, jnp.int32)
                for q in range(dim // lanes):
                    jidx = lane_iota + q * lanes
                    vals = plsc.load_gather(win.at[buf, r], [c, jidx])
                    plsc.store_scatter(stgT, [jidx, k], vals)
                return 0
            lax.fori_loop(0, grp, ex_body, 0)

        fire(0, 0)
        for g in range(n_grp):
            if g + 1 < n_grp:
                fire(g + 1, (g + 1) % 2)
            drain(g % 2)
            extract(g, g % 2)
        pltpu.sync_copy(stgT, outT_hbm.at[:, pl.ds(base, b_per_w)])

    outT = body(x, table)
    return outT.T


def kernel(x, table):
    return _lookup(x.astype(jnp.int32), table)
